# Initial kernel scaffold; baseline (speedup 1.0000x reference)
import jax
import jax.numpy as jnp
from jax.experimental import pallas as pl


def kernel(ids, weight, alpha):
    raise NotImplementedError("write your pallas kernel here")


# speedup vs baseline: 4.5059x; 4.5059x over previous
import functools

import jax
import jax.numpy as jnp
from jax import lax
from jax.experimental import pallas as pl
from jax.experimental.pallas import tpu as pltpu
from jax.experimental.pallas import tpu_sc as plsc

B, L, D = 1024, 200, 128
NC, NS, LANES = 2, 16, 16
NW = NC * NS
ROWS_PER_W = B // NW
CHUNKS = ((0, 128), (128, L - 128))


def _sc_kernel(ids_hbm, weight_hbm, alpha_hbm, out_hbm, idx_v, x_v, alpha_v, sem):
    wid = lax.axis_index("s") * NC + lax.axis_index("c")
    pltpu.sync_copy(alpha_hbm, alpha_v)
    a = alpha_v[...]

    def row_body(r, carry):
        b = wid * ROWS_PER_W + r
        pltpu.sync_copy(ids_hbm.at[b], idx_v)
        for off, n in CHUNKS:
            pltpu.async_copy(
                weight_hbm.at[idx_v.at[pl.ds(off, n)]],
                x_v.at[pl.ds(off, n)],
                sem,
            ).wait()

        def mix_body(i, carry2):
            l = (L - 1) - i
            for j in range(D // LANES):
                sl = pl.ds(j * LANES, LANES)
                x_v[l, sl] = x_v[l, sl] + a * x_v[l - 1, sl]
            return carry2

        lax.fori_loop(0, L - 1, mix_body, 0)
        pltpu.sync_copy(x_v, out_hbm.at[b])
        return carry

    lax.fori_loop(0, ROWS_PER_W, row_body, 0)


@jax.jit
def kernel(ids, weight, alpha):
    alpha_vec = jnp.broadcast_to(alpha.astype(jnp.float32), (LANES,))
    mesh = plsc.VectorSubcoreMesh(core_axis_name="c", subcore_axis_name="s")
    run = pl.kernel(
        _sc_kernel,
        out_type=jax.ShapeDtypeStruct((B, L, D), jnp.float32),
        mesh=mesh,
        scratch_types=[
            pltpu.VMEM((L,), jnp.int32),
            pltpu.VMEM((L, D), jnp.float32),
            pltpu.VMEM((LANES,), jnp.float32),
            pltpu.SemaphoreType.DMA,
        ],
    )
    return run(ids.astype(jnp.int32), weight, alpha_vec)
